# baseline (speedup 1.0000x reference)
import functools

import jax
import jax.numpy as jnp
from jax import lax
from jax.experimental import pallas as pl
from jax.experimental.pallas import tpu as pltpu
from jax.experimental.pallas import tpu_sc as plsc

NC, NS, L = 2, 16, 16
NW = NC * NS
R, C = 128, 32768
K = 16
HC = C // 2
BR = 8
NCH = 4
CW = HC // NCH

_MESH = plsc.VectorSubcoreMesh(
    core_axis_name="c", subcore_axis_name="s", num_cores=NC, num_subcores=NS
)

_KEY_BASE = 0x3E19999A


def _tc_score_body(x_ref, out_ref):
    x = x_ref[...]
    s = jnp.sum(x, axis=1)
    q = jnp.sum(x * x, axis=1)
    var = (q - s * s * (1.0 / C)) * (1.0 / (C - 1))
    std = jnp.sqrt(jnp.maximum(var, 0.0))
    realism = jnp.where(
        std < 0.01,
        std * 10.0,
        jnp.where(std > 0.5, 0.5 / std, 1.0 - jnp.abs(std - 0.1)),
    )
    out_ref[...] = (realism + 0.15).reshape(1, 1, BR)


_score_tc = pl.pallas_call(
    _tc_score_body,
    grid=(R // BR,),
    in_specs=[pl.BlockSpec((BR, C), lambda i: (i, 0))],
    out_specs=pl.BlockSpec((1, 1, BR), lambda i: (i, 0, 0)),
    out_shape=jax.ShapeDtypeStruct((R // BR, 1, BR), jnp.float32),
    compiler_params=pltpu.CompilerParams(dimension_semantics=("arbitrary",)),
)


def _lane_iota():
    return lax.iota(jnp.int32, L)


def _gather_scores(sraw, rows):
    return plsc.load_gather(
        sraw,
        [
            lax.shift_right_arithmetic(rows, jnp.full((L,), 3, jnp.int32)),
            jnp.full((L,), 0, jnp.int32),
            lax.bitwise_and(rows, jnp.full((L,), BR - 1, jnp.int32)),
        ],
    )


@functools.partial(
    pl.kernel,
    out_type=(
        jax.ShapeDtypeStruct((K, C), jnp.float32),
        jax.ShapeDtypeStruct((K,), jnp.float32),
    ),
    mesh=_MESH,
    scratch_types=[
        pltpu.VMEM((R // BR, 1, BR), jnp.float32),
        pltpu.VMEM((K,), jnp.int32),
        pltpu.VMEM((K,), jnp.float32),
        pltpu.VMEM((1,), jnp.int32),
        pltpu.VMEM((1, HC), jnp.float32),
        pltpu.SemaphoreType.DMA,
        pltpu.SemaphoreType.DMA,
        pltpu.SemaphoreType.DMA,
    ],
    compiler_params=pltpu.CompilerParams(needs_layout_passes=False),
)
def _topk_gather_stage(chunks_hbm, scores_hbm, out_hbm, oscores_hbm,
                       sraw, tidx, tsc, idxv, halfbuf, semA, semB, semO):
    wid = lax.axis_index("s") * NC + lax.axis_index("c")
    lane = _lane_iota()
    zero = jnp.full((L,), 0, jnp.int32)

    pltpu.sync_copy(scores_hbm, sraw)

    pairs = []
    for v in range(8):
        jv = lane + (16 * v)
        sv = _gather_scores(sraw, jv)
        bits = lax.bitcast_convert_type(sv, jnp.uint32)
        diff = bits - jnp.full((L,), _KEY_BASE, jnp.uint32)
        key = lax.bitwise_or(
            lax.shift_left(diff, jnp.full((L,), 7, jnp.uint32)),
            lax.bitcast_convert_type(jnp.full((L,), 127, jnp.int32) - jv,
                                     jnp.uint32),
        )
        pairs.append(plsc.sort_key_val(key, jv, descending=True))

    def merge(a, b):
        ka, va = a
        kb, vb = b
        kr = lax.rev(kb, (0,))
        vr = lax.rev(vb, (0,))
        m = ka >= kr
        kk = jnp.where(m, ka, kr)
        vv = jnp.where(m, va, vr)
        return plsc.sort_key_val(kk, vv, descending=True)

    while len(pairs) > 1:
        pairs = [merge(pairs[i], pairs[i + 1]) for i in range(0, len(pairs), 2)]
    _, top_rows = pairs[0]

    tidx[...] = top_rows
    tsc[...] = _gather_scores(sraw, top_rows)

    @pl.when(wid == 0)
    def _():
        pltpu.sync_copy(tsc, oscores_hbm)

    r = wid // 2
    h = wid % 2
    rowvec = plsc.load_gather(tidx, [jnp.full((L,), r, jnp.int32)])
    plsc.store_scatter(idxv, [zero], rowvec, mask=lane == 0)
    colbase = h * HC
    sems = (semA, semB)
    gin = [None] * NCH
    gout = [None] * NCH
    gin[0] = pltpu.async_copy(
        chunks_hbm.at[idxv, pl.ds(colbase, CW)], halfbuf.at[:, pl.ds(0, CW)], semA
    )
    for k in range(NCH):
        if k + 1 < NCH:
            off = (k + 1) * CW
            gin[k + 1] = pltpu.async_copy(
                chunks_hbm.at[idxv, pl.ds(colbase + off, CW)],
                halfbuf.at[:, pl.ds(off, CW)],
                sems[(k + 1) % 2],
            )
        gin[k].wait()
        gout[k] = pltpu.async_copy(
            halfbuf.at[:, pl.ds(k * CW, CW)],
            out_hbm.at[pl.ds(r, 1), pl.ds(colbase + k * CW, CW)],
            semO,
        )
    for k in range(NCH):
        gout[k].wait()


def kernel(chunks, regime_probs, keep_top_k):
    del regime_probs, keep_top_k
    scores = _score_tc(chunks)
    return _topk_gather_stage(chunks, scores)


# speedup vs baseline: 1.2332x; 1.2332x over previous
import functools

import jax
import jax.numpy as jnp
from jax import lax
from jax.experimental import pallas as pl
from jax.experimental.pallas import tpu as pltpu
from jax.experimental.pallas import tpu_sc as plsc

NC, NS, L = 2, 16, 16
NW = NC * NS
R, C = 128, 32768
K = 16
HC = C // 2
BR = 32

_MESH = plsc.VectorSubcoreMesh(
    core_axis_name="c", subcore_axis_name="s", num_cores=NC, num_subcores=NS
)

_KEY_BASE = 0x3E19999A


def _tc_score_body(x_ref, out_ref):
    i = pl.program_id(0)
    x = x_ref[...]
    s = jnp.sum(x, axis=1)
    q = jnp.sum(x * x, axis=1)
    var = (q - s * s * (1.0 / C)) * (1.0 / (C - 1))
    std = jnp.sqrt(jnp.maximum(var, 0.0))
    realism = jnp.where(
        std < 0.01,
        std * 10.0,
        jnp.where(std > 0.5, 0.5 / std, 1.0 - jnp.abs(std - 0.1)),
    )
    out_ref[pl.ds(i, 1)] = (realism + 0.15).reshape(1, 1, BR)


_score_tc = pl.pallas_call(
    _tc_score_body,
    grid=(R // BR,),
    in_specs=[pl.BlockSpec((BR, C), lambda i: (i, 0))],
    out_specs=pl.BlockSpec((R // BR, 1, BR), lambda i: (0, 0, 0)),
    out_shape=jax.ShapeDtypeStruct((R // BR, 1, BR), jnp.float32),
    compiler_params=pltpu.CompilerParams(dimension_semantics=("arbitrary",)),
)


def _lane_iota():
    return lax.iota(jnp.int32, L)


def _gather_scores(sraw, rows):
    return plsc.load_gather(
        sraw,
        [
            lax.shift_right_arithmetic(rows, jnp.full((L,), 5, jnp.int32)),
            jnp.full((L,), 0, jnp.int32),
            lax.bitwise_and(rows, jnp.full((L,), BR - 1, jnp.int32)),
        ],
    )


@functools.partial(
    pl.kernel,
    out_type=(
        jax.ShapeDtypeStruct((K, C), jnp.float32),
        jax.ShapeDtypeStruct((K,), jnp.float32),
    ),
    mesh=_MESH,
    scratch_types=[
        pltpu.VMEM((R // BR, 1, BR), jnp.float32),
        pltpu.VMEM((K,), jnp.int32),
        pltpu.VMEM((K,), jnp.float32),
        pltpu.VMEM((1,), jnp.int32),
        pltpu.VMEM((1, HC), jnp.float32),
        pltpu.SemaphoreType.DMA,
        pltpu.SemaphoreType.DMA,
        pltpu.SemaphoreType.DMA,
    ],
    compiler_params=pltpu.CompilerParams(needs_layout_passes=False),
)
def _topk_gather_stage(chunks_hbm, scores_hbm, out_hbm, oscores_hbm,
                       sraw, tidx, tsc, idxv, halfbuf, semA, semB, semO):
    wid = lax.axis_index("s") * NC + lax.axis_index("c")
    lane = _lane_iota()
    zero = jnp.full((L,), 0, jnp.int32)

    pltpu.sync_copy(scores_hbm, sraw)

    pairs = []
    for v in range(8):
        jv = lane + (16 * v)
        sv = _gather_scores(sraw, jv)
        bits = lax.bitcast_convert_type(sv, jnp.uint32)
        diff = bits - jnp.full((L,), _KEY_BASE, jnp.uint32)
        key = lax.bitwise_or(
            lax.shift_left(diff, jnp.full((L,), 7, jnp.uint32)),
            lax.bitcast_convert_type(jnp.full((L,), 127, jnp.int32) - jv,
                                     jnp.uint32),
        )
        pairs.append(plsc.sort_key_val(key, jv, descending=True))

    def merge(a, b):
        ka, va = a
        kb, vb = b
        kr = lax.rev(kb, (0,))
        vr = lax.rev(vb, (0,))
        m = ka >= kr
        kk = jnp.where(m, ka, kr)
        vv = jnp.where(m, va, vr)
        return plsc.sort_key_val(kk, vv, descending=True)

    while len(pairs) > 1:
        pairs = [merge(pairs[i], pairs[i + 1]) for i in range(0, len(pairs), 2)]
    _, top_rows = pairs[0]

    tidx[...] = top_rows
    tsc[...] = _gather_scores(sraw, top_rows)

    @pl.when(wid == 0)
    def _():
        pltpu.sync_copy(tsc, oscores_hbm)

    r = wid // 2
    h = wid % 2
    rowvec = plsc.load_gather(tidx, [jnp.full((L,), r, jnp.int32)])
    plsc.store_scatter(idxv, [zero], rowvec, mask=lane == 0)
    colbase = h * HC
    pltpu.async_copy(chunks_hbm.at[idxv, pl.ds(colbase, HC)], halfbuf, semA).wait()
    pltpu.sync_copy(halfbuf, out_hbm.at[pl.ds(r, 1), pl.ds(colbase, HC)])


def kernel(chunks, regime_probs, keep_top_k):
    del regime_probs, keep_top_k
    scores = _score_tc(chunks)
    return _topk_gather_stage(chunks, scores)
